# baseline (speedup 1.0000x reference)
import jax
import jax.numpy as jnp
from jax import lax
from jax.experimental import pallas as pl
from jax.experimental.pallas import tpu as pltpu
from jax.experimental.pallas import tpu_sc as plsc

N = 100000
D = 512
S = 15000

L = 16
NC = 2
NS = 16
NW = NC * NS
NPAD = 100352
PAD_IDX = NPAD - 1

ROWS_BLK = 5000

KB = 128
KCH = 4
S_SCAT = NW * KCH * KB


def _scatter_sc_body(seeds_hbm, mask_ref, idx_v, zeros_v):
    wid = lax.axis_index("s") * NC + lax.axis_index("c")
    pltpu.sync_copy(seeds_hbm.at[wid], idx_v)

    zeros = jnp.zeros((L,), jnp.float32)

    def init(i, c):
        zeros_v[pl.ds(i * L, L)] = zeros
        return c
    lax.fori_loop(0, KB // L, init, 0)

    for j in range(KCH):
        pltpu.sync_copy(zeros_v, mask_ref.at[idx_v.at[j]])


def _build_mask(seeds3d):
    mesh = plsc.VectorSubcoreMesh(core_axis_name="c", subcore_axis_name="s")
    mask_ref = jax.new_ref(jnp.ones((NPAD,), jnp.float32))
    pl.kernel(
        _scatter_sc_body,
        mesh=mesh,
        out_type=(),
        scratch_types=[
            pltpu.VMEM((KCH, KB), jnp.int32),
            pltpu.VMEM((KB,), jnp.float32),
        ],
        compiler_params=pltpu.CompilerParams(needs_layout_passes=False),
    )(seeds3d, mask_ref)
    return mask_ref[...]


def _blend_body(emb_ref, m_ref, tok_ref, out_ref):
    m = m_ref[...]
    out_ref[...] = jnp.where(m == 0.0, tok_ref[...], emb_ref[...])


def kernel(embeds, seeds, mask_token):
    seeds_padded = jnp.concatenate(
        [seeds.astype(jnp.int32),
         jnp.full((S_SCAT - S,), PAD_IDX, jnp.int32)])
    mask = _build_mask(seeds_padded.reshape(NW, KCH, KB))
    mask2d = mask.reshape(NPAD, 1)

    out = pl.pallas_call(
        _blend_body,
        grid=(N // ROWS_BLK,),
        in_specs=[
            pl.BlockSpec((ROWS_BLK, D), lambda i: (i, 0)),
            pl.BlockSpec((ROWS_BLK, 1), lambda i: (i, 0)),
            pl.BlockSpec((1, D), lambda i: (0, 0)),
        ],
        out_specs=pl.BlockSpec((ROWS_BLK, D), lambda i: (i, 0)),
        out_shape=jax.ShapeDtypeStruct((N, D), jnp.float32),
    )(embeds, mask2d, mask_token)
    return (out, seeds)


# speedup vs baseline: 2.1416x; 2.1416x over previous
import functools

import jax
import jax.numpy as jnp
from jax import lax
from jax.experimental import pallas as pl
from jax.experimental.pallas import tpu as pltpu
from jax.experimental.pallas import tpu_sc as plsc

N = 100000
D = 512
S = 15000

L = 16
NC = 2
NS = 16
NW = NC * NS
CHUNK = 3136
NPAD = NW * CHUNK
S_PAD = ((S + L - 1) // L) * L
PAD_IDX = NPAD - 1

ROWS_BLK = 5000


def _mask_sc_body(seeds_hbm, mask_hbm, seeds_v, mask_v):
    wid = lax.axis_index("s") * NC + lax.axis_index("c")
    base = wid * CHUNK
    pltpu.sync_copy(seeds_hbm, seeds_v)

    ones = jnp.ones((L,), jnp.float32)
    zeros = jnp.zeros((L,), jnp.float32)

    @plsc.parallel_loop(0, CHUNK, L, unroll=8)
    def _init(i):
        mask_v[pl.ds(i, L)] = ones

    @plsc.parallel_loop(0, S_PAD, L, unroll=8)
    def _scan(g):
        s = seeds_v[pl.ds(g, L)]
        local = s - base
        inr = (local >= 0) & (local < CHUNK)
        idx = jnp.where(inr, local, 0)
        plsc.store_scatter(mask_v, [idx], zeros, mask=inr)

    pltpu.sync_copy(mask_v, mask_hbm.at[pl.ds(base, CHUNK)])


def _build_mask(seeds_padded):
    mesh = plsc.VectorSubcoreMesh(core_axis_name="c", subcore_axis_name="s")
    return pl.kernel(
        _mask_sc_body,
        mesh=mesh,
        out_type=jax.ShapeDtypeStruct((NPAD,), jnp.float32),
        scratch_types=[
            pltpu.VMEM((S_PAD,), jnp.int32),
            pltpu.VMEM((CHUNK,), jnp.float32),
        ],
        compiler_params=pltpu.CompilerParams(needs_layout_passes=False),
    )(seeds_padded)


def _blend_body(emb_ref, m_ref, tok_ref, out_ref):
    m = m_ref[...]
    out_ref[...] = jnp.where(m == 0.0, tok_ref[...], emb_ref[...])


def kernel(embeds, seeds, mask_token):
    seeds_padded = jnp.concatenate(
        [seeds.astype(jnp.int32),
         jnp.full((S_PAD - S,), PAD_IDX, jnp.int32)])
    mask = _build_mask(seeds_padded)
    mask2d = mask.reshape(NPAD, 1)

    out = pl.pallas_call(
        _blend_body,
        grid=(N // ROWS_BLK,),
        in_specs=[
            pl.BlockSpec((ROWS_BLK, D), lambda i: (i, 0)),
            pl.BlockSpec((ROWS_BLK, 1), lambda i: (i, 0)),
            pl.BlockSpec((1, D), lambda i: (0, 0)),
        ],
        out_specs=pl.BlockSpec((ROWS_BLK, D), lambda i: (i, 0)),
        out_shape=jax.ShapeDtypeStruct((N, D), jnp.float32),
    )(embeds, mask2d, mask_token)
    return (out, seeds)


# speedup vs baseline: 2.1452x; 1.0017x over previous
import functools

import jax
import jax.numpy as jnp
from jax import lax
from jax.experimental import pallas as pl
from jax.experimental.pallas import tpu as pltpu
from jax.experimental.pallas import tpu_sc as plsc

N = 100000
D = 512
S = 15000

L = 16
NC = 2
NS = 16
NW = NC * NS
CHUNK = 3136
NPAD = NW * CHUNK
S_FULL = (S // L) * L

ROWS_BLK = 5000


def _mask_sc_body(seeds_hbm, mask_hbm, seeds_v, mask_v):
    wid = lax.axis_index("s") * NC + lax.axis_index("c")
    base = wid * CHUNK
    pltpu.sync_copy(seeds_hbm, seeds_v)

    ones = jnp.ones((L,), jnp.float32)
    zeros = jnp.zeros((L,), jnp.float32)

    @plsc.parallel_loop(0, CHUNK, L, unroll=8)
    def _init(i):
        mask_v[pl.ds(i, L)] = ones

    def _scatter_group(g):
        s = seeds_v[pl.ds(g, L)]
        local = s - base
        inr = (local >= 0) & (local < CHUNK)
        idx = jnp.where(inr, local, 0)
        plsc.store_scatter(mask_v, [idx], zeros, mask=inr)

    @plsc.parallel_loop(0, S_FULL, L, unroll=8)
    def _scan(g):
        _scatter_group(g)

    _scatter_group(S - L)

    pltpu.sync_copy(mask_v, mask_hbm.at[pl.ds(base, CHUNK)])


def _build_mask(seeds_padded):
    mesh = plsc.VectorSubcoreMesh(core_axis_name="c", subcore_axis_name="s")
    return pl.kernel(
        _mask_sc_body,
        mesh=mesh,
        out_type=jax.ShapeDtypeStruct((NPAD,), jnp.float32),
        scratch_types=[
            pltpu.VMEM((S,), jnp.int32),
            pltpu.VMEM((CHUNK,), jnp.float32),
        ],
        compiler_params=pltpu.CompilerParams(needs_layout_passes=False),
    )(seeds_padded)


def _blend_body(emb_ref, m_ref, tok_ref, out_ref):
    m = m_ref[...]
    out_ref[...] = jnp.where(m == 0.0, tok_ref[...], emb_ref[...])


def kernel(embeds, seeds, mask_token):
    mask = _build_mask(seeds)
    mask2d = mask.reshape(NPAD, 1)

    out = pl.pallas_call(
        _blend_body,
        grid=(N // ROWS_BLK,),
        in_specs=[
            pl.BlockSpec((ROWS_BLK, D), lambda i: (i, 0)),
            pl.BlockSpec((ROWS_BLK, 1), lambda i: (i, 0)),
            pl.BlockSpec((1, D), lambda i: (0, 0)),
        ],
        out_specs=pl.BlockSpec((ROWS_BLK, D), lambda i: (i, 0)),
        out_shape=jax.ShapeDtypeStruct((N, D), jnp.float32),
    )(embeds, mask2d, mask_token)
    return (out, seeds)


# speedup vs baseline: 2.1458x; 1.0003x over previous
import functools

import jax
import jax.numpy as jnp
from jax import lax
from jax.experimental import pallas as pl
from jax.experimental.pallas import tpu as pltpu
from jax.experimental.pallas import tpu_sc as plsc

N = 100000
D = 512
S = 15000

L = 16
NC = 2
NS = 16
NW = NC * NS
CHUNK = 3136
NPAD = NW * CHUNK
S_FULL = (S // L) * L

ROWS_BLK = 4000


def _mask_sc_body(seeds_hbm, mask_hbm, seeds_v, mask_v):
    wid = lax.axis_index("s") * NC + lax.axis_index("c")
    base = wid * CHUNK
    pltpu.sync_copy(seeds_hbm, seeds_v)

    ones = jnp.ones((L,), jnp.float32)
    zeros = jnp.zeros((L,), jnp.float32)

    @plsc.parallel_loop(0, CHUNK, L, unroll=8)
    def _init(i):
        mask_v[pl.ds(i, L)] = ones

    def _scatter_group(g):
        s = seeds_v[pl.ds(g, L)]
        local = s - base
        inr = (local >= 0) & (local < CHUNK)
        idx = jnp.where(inr, local, 0)
        plsc.store_scatter(mask_v, [idx], zeros, mask=inr)

    @plsc.parallel_loop(0, S_FULL, L, unroll=8)
    def _scan(g):
        _scatter_group(g)

    _scatter_group(S - L)

    pltpu.sync_copy(mask_v, mask_hbm.at[pl.ds(base, CHUNK)])


def _build_mask(seeds_padded):
    mesh = plsc.VectorSubcoreMesh(core_axis_name="c", subcore_axis_name="s")
    return pl.kernel(
        _mask_sc_body,
        mesh=mesh,
        out_type=jax.ShapeDtypeStruct((NPAD,), jnp.float32),
        scratch_types=[
            pltpu.VMEM((S,), jnp.int32),
            pltpu.VMEM((CHUNK,), jnp.float32),
        ],
        compiler_params=pltpu.CompilerParams(needs_layout_passes=False),
    )(seeds_padded)


def _blend_body(emb_ref, m_ref, tok_ref, out_ref):
    m = m_ref[...]
    out_ref[...] = jnp.where(m == 0.0, tok_ref[...], emb_ref[...])


def kernel(embeds, seeds, mask_token):
    mask = _build_mask(seeds)
    mask2d = mask.reshape(NPAD, 1)

    out = pl.pallas_call(
        _blend_body,
        grid=(N // ROWS_BLK,),
        in_specs=[
            pl.BlockSpec((ROWS_BLK, D), lambda i: (i, 0)),
            pl.BlockSpec((ROWS_BLK, 1), lambda i: (i, 0)),
            pl.BlockSpec((1, D), lambda i: (0, 0)),
        ],
        out_specs=pl.BlockSpec((ROWS_BLK, D), lambda i: (i, 0)),
        out_shape=jax.ShapeDtypeStruct((N, D), jnp.float32),
    )(embeds, mask2d, mask_token)
    return (out, seeds)


# speedup vs baseline: 2.7236x; 1.2693x over previous
import functools

import jax
import jax.numpy as jnp
from jax import lax
from jax.experimental import pallas as pl
from jax.experimental.pallas import tpu as pltpu
from jax.experimental.pallas import tpu_sc as plsc

N = 100000
D = 512
S = 15000

L = 16
NC = 2
NS = 16
NW = NC * NS
CHUNK = 3200
NPAD = NW * CHUNK
S_FULL = (S // L) * L

ROWS_BLK = 4096
NBLK = (N + ROWS_BLK - 1) // ROWS_BLK


def _mask_sc_body(seeds_hbm, mask_hbm, seeds_v, mask_v):
    wid = lax.axis_index("s") * NC + lax.axis_index("c")
    base = wid * CHUNK
    pltpu.sync_copy(seeds_hbm, seeds_v)

    ones = jnp.ones((L,), jnp.float32)
    zeros = jnp.zeros((L,), jnp.float32)

    @plsc.parallel_loop(0, CHUNK, L, unroll=8)
    def _init(i):
        mask_v[pl.ds(i, L)] = ones

    def _scatter_group(g):
        s = seeds_v[pl.ds(g, L)]
        local = s - base
        inr = (local >= 0) & (local < CHUNK)
        idx = jnp.where(inr, local, 0)
        plsc.store_scatter(mask_v, [idx], zeros, mask=inr)

    @plsc.parallel_loop(0, S_FULL, L, unroll=8)
    def _scan(g):
        _scatter_group(g)

    _scatter_group(S - L)

    pltpu.sync_copy(mask_v, mask_hbm.at[pl.ds(base, CHUNK)])


def _build_mask(seeds_padded):
    mesh = plsc.VectorSubcoreMesh(core_axis_name="c", subcore_axis_name="s")
    return pl.kernel(
        _mask_sc_body,
        mesh=mesh,
        out_type=jax.ShapeDtypeStruct((NPAD,), jnp.float32),
        scratch_types=[
            pltpu.VMEM((S,), jnp.int32),
            pltpu.VMEM((CHUNK,), jnp.float32),
        ],
        compiler_params=pltpu.CompilerParams(needs_layout_passes=False),
    )(seeds_padded)


def _blend_body(emb_ref, m_ref, tok_ref, out_ref):
    i = pl.program_id(0)
    m = m_ref[pl.ds(i * ROWS_BLK, ROWS_BLK)].reshape(ROWS_BLK, 1)
    out_ref[...] = jnp.where(m == 0.0, tok_ref[...], emb_ref[...])


def kernel(embeds, seeds, mask_token):
    mask = _build_mask(seeds)

    out = pl.pallas_call(
        _blend_body,
        grid=(NBLK,),
        in_specs=[
            pl.BlockSpec((ROWS_BLK, D), lambda i: (i, 0)),
            pl.BlockSpec((NPAD,), lambda i: (0,)),
            pl.BlockSpec((1, D), lambda i: (0, 0)),
        ],
        out_specs=pl.BlockSpec((ROWS_BLK, D), lambda i: (i, 0)),
        out_shape=jax.ShapeDtypeStruct((N, D), jnp.float32),
    )(embeds, mask, mask_token)
    return (out, seeds)
